# baseline (speedup 1.0000x reference)
import functools

import jax
import jax.numpy as jnp
from jax.experimental import pallas as pl
from jax.experimental.pallas import tpu as pltpu

_ETYPE = jnp.bfloat16


def _dinv_body(e_ref, out_ref, ec_ref, *, ns):
    s = pl.program_id(0)

    @pl.when(s == 0)
    def _():
        out_ref[...] = jnp.zeros_like(out_ref)

    e = e_ref[...]
    out_ref[...] += jnp.sum(e, axis=0, keepdims=True)
    ec_ref[...] = e.astype(_ETYPE)

    @pl.when(s == ns - 1)
    def _():
        out_ref[...] = jax.lax.rsqrt(out_ref[...] + 1.0)


def _colsum_dinv(e, bs):
    n = e.shape[0]
    ns = n // bs
    return pl.pallas_call(
        functools.partial(_dinv_body, ns=ns),
        grid=(ns,),
        in_specs=[pl.BlockSpec((bs, n), lambda s: (s, 0))],
        out_specs=[
            pl.BlockSpec((1, n), lambda s: (0, 0)),
            pl.BlockSpec((bs, n), lambda s: (s, 0)),
        ],
        out_shape=[
            jax.ShapeDtypeStruct((1, n), jnp.float32),
            jax.ShapeDtypeStruct((n, n), _ETYPE),
        ],
    )(e)


def _gcn_body(
    wt_ref, xin_ref, e_ref, dinv_s_ref, dinv_d_ref, b_ref, out_ref,
    xa_ref, xb_ref, acc_ref, ybuf_ref,
    *, n, nl, nd, ns, bs, bd,
):
    l, d, s = pl.program_id(0), pl.program_id(1), pl.program_id(2)
    row_lim = n - s * bs

    wt = wt_ref[0].astype(_ETYPE)

    def proj_layer1():
        rows = jax.lax.broadcasted_iota(jnp.int32, xin_ref.shape, 0)
        x = jnp.where(rows < row_lim, xin_ref[...], 0.0).astype(_ETYPE)
        return jax.lax.dot_general(
            wt, x, (((1,), (1,)), ((), ())), preferred_element_type=jnp.float32
        )

    def proj_resident(buf_ref):
        def f():
            x = buf_ref[:, pl.ds(s * bs, bs)].astype(_ETYPE)
            return jnp.dot(wt, x, preferred_element_type=jnp.float32)

        return f

    p = jax.lax.cond(
        l == 0,
        proj_layer1,
        lambda: jax.lax.cond(
            l == 1, proj_resident(xa_ref), proj_resident(xb_ref)
        ),
    )
    y = (p * dinv_s_ref[...]).astype(_ETYPE)

    @pl.when(s == 0)
    def _():
        acc_ref[...] = jnp.zeros_like(acc_ref)

    r_ds = bd // bs

    @pl.when((s >= d * r_ds) & (s < (d + 1) * r_ds))
    def _():
        ybuf_ref[:, pl.ds((s - d * r_ds) * bs, bs)] = y

    @pl.when(row_lim >= bs)
    def _():
        acc_ref[...] += jnp.dot(y, e_ref[...], preferred_element_type=jnp.float32)

    @pl.when(row_lim < bs)
    def _():
        erows = jax.lax.broadcasted_iota(jnp.int32, e_ref.shape, 0)
        e = jnp.where(erows < row_lim, e_ref[...], jnp.zeros((), e_ref.dtype))
        acc_ref[...] += jnp.dot(y, e, preferred_element_type=jnp.float32)

    @pl.when(s == ns - 1)
    def _():
        yself = ybuf_ref[...].astype(jnp.float32)
        r = dinv_d_ref[...] * (acc_ref[...] + yself) + b_ref[0]
        r = jnp.where(l < nl - 1, jnp.maximum(r, 0.0), r)

        lanes = jax.lax.broadcasted_iota(jnp.int32, r.shape, 1)
        r_masked = jnp.where(lanes < n - d * bd, r, 0.0)

        @pl.when(l == 0)
        def _():
            xa_ref[:, pl.ds(d * bd, bd)] = r_masked

        @pl.when(l == 1)
        def _():
            xb_ref[:, pl.ds(d * bd, bd)] = r_masked

        @pl.when(l == nl - 1)
        def _():
            out_ref[...] = r.T


def _gcn_layers(wstack, bstack, nf, e_c, dinv_p, bs, bd):
    nl, h = wstack.shape[0], wstack.shape[1]
    n = e_c.shape[0]
    np_ = dinv_p.shape[1]
    nd, ns = np_ // bd, np_ // bs
    return pl.pallas_call(
        functools.partial(
            _gcn_body, n=n, nl=nl, nd=nd, ns=ns, bs=bs, bd=bd
        ),
        grid=(nl, nd, ns),
        in_specs=[
            pl.BlockSpec((1, h, h), lambda l, d, s: (l, 0, 0)),
            pl.BlockSpec(
                (bs, h), lambda l, d, s: (jnp.where(l == 0, s, 0), 0)
            ),
            pl.BlockSpec((bs, bd), lambda l, d, s: (s, d)),
            pl.BlockSpec((1, bs), lambda l, d, s: (0, s)),
            pl.BlockSpec((1, bd), lambda l, d, s: (0, d)),
            pl.BlockSpec((1, h, 1), lambda l, d, s: (l, 0, 0)),
        ],
        out_specs=pl.BlockSpec((bd, h), lambda l, d, s: (d, 0)),
        out_shape=jax.ShapeDtypeStruct((n, h), jnp.float32),
        scratch_shapes=[
            pltpu.VMEM((h, np_), jnp.float32),
            pltpu.VMEM((h, np_), jnp.float32),
            pltpu.VMEM((h, bd), jnp.float32),
            pltpu.VMEM((h, bd), _ETYPE),
        ],
    )(wstack, nf, e_c, dinv_p, dinv_p, bstack)


def kernel(node_features, edges, W1, b1, W2, b2, W3, b3):
    n, dim = node_features.shape
    bs = min(1280, -(-n // 128) * 128)
    np_ = -(-n // bs) * bs
    bd = min(4 * bs, np_)

    csum_bs = 1
    for c in range(16, min(n, 80) + 1, 16):
        if n % c == 0:
            csum_bs = c
    dinv, e_c = _colsum_dinv(edges, bs=csum_bs)

    dinv_p = jnp.pad(dinv, ((0, 0), (0, np_ - n)))
    wstack = jnp.stack([W1.T, W2.T, W3.T])
    bstack = jnp.stack([b1, b2, b3]).reshape(3, -1, 1)
    return _gcn_layers(wstack, bstack, node_features, e_c, dinv_p, bs, bd)


# speedup vs baseline: 1.0333x; 1.0333x over previous
import functools

import jax
import jax.numpy as jnp
from jax.experimental import pallas as pl
from jax.experimental.pallas import tpu as pltpu

_ETYPE = jnp.bfloat16


def _dinv_body(e_ref, out_ref, ec_ref, *, ns):
    s = pl.program_id(0)

    @pl.when(s == 0)
    def _():
        out_ref[...] = jnp.zeros_like(out_ref)

    e = e_ref[...]
    out_ref[...] += jnp.sum(e, axis=0, keepdims=True)
    ec_ref[...] = e.astype(_ETYPE)

    @pl.when(s == ns - 1)
    def _():
        out_ref[...] = jax.lax.rsqrt(out_ref[...] + 1.0)


def _colsum_dinv(e, bs):
    n = e.shape[0]
    ns = n // bs
    return pl.pallas_call(
        functools.partial(_dinv_body, ns=ns),
        grid=(ns,),
        in_specs=[pl.BlockSpec((bs, n), lambda s: (s, 0))],
        out_specs=[
            pl.BlockSpec((1, n), lambda s: (0, 0)),
            pl.BlockSpec((bs, n), lambda s: (s, 0)),
        ],
        out_shape=[
            jax.ShapeDtypeStruct((1, n), jnp.float32),
            jax.ShapeDtypeStruct((n, n), _ETYPE),
        ],
    )(e)


def _gcn_body(
    wt_ref, xin_ref, e_ref, dinv_s_ref, dinv_d_ref, b_ref, out_ref,
    xa_ref, xb_ref, acc_ref, ybuf_ref,
    *, n, nl, nd, ns, bs, bd,
):
    l, d, s = pl.program_id(0), pl.program_id(1), pl.program_id(2)
    row_lim = n - s * bs

    wt = wt_ref[0].astype(_ETYPE)

    def proj_layer1():
        rows = jax.lax.broadcasted_iota(jnp.int32, xin_ref.shape, 0)
        x = jnp.where(rows < row_lim, xin_ref[...], 0.0).astype(_ETYPE)
        return jax.lax.dot_general(
            wt, x, (((1,), (1,)), ((), ())), preferred_element_type=jnp.float32
        )

    def proj_resident(buf_ref):
        def f():
            x = buf_ref[:, pl.ds(s * bs, bs)].astype(_ETYPE)
            return jnp.dot(wt, x, preferred_element_type=jnp.float32)

        return f

    p = jax.lax.cond(
        l == 0,
        proj_layer1,
        lambda: jax.lax.cond(
            l == 1, proj_resident(xa_ref), proj_resident(xb_ref)
        ),
    )
    y = (p * dinv_s_ref[...]).astype(_ETYPE)

    @pl.when(s == 0)
    def _():
        acc_ref[...] = jnp.zeros_like(acc_ref)

    r_ds = bd // bs

    @pl.when((s >= d * r_ds) & (s < (d + 1) * r_ds))
    def _():
        ybuf_ref[:, pl.ds((s - d * r_ds) * bs, bs)] = y

    @pl.when(row_lim >= bs)
    def _():
        acc_ref[...] += jnp.dot(y, e_ref[...], preferred_element_type=jnp.float32)

    @pl.when(row_lim < bs)
    def _():
        erows = jax.lax.broadcasted_iota(jnp.int32, e_ref.shape, 0)
        e = jnp.where(erows < row_lim, e_ref[...], jnp.zeros((), e_ref.dtype))
        acc_ref[...] += jnp.dot(y, e, preferred_element_type=jnp.float32)

    @pl.when(s == ns - 1)
    def _():
        yself = ybuf_ref[...].astype(jnp.float32)
        r = dinv_d_ref[...] * (acc_ref[...] + yself) + b_ref[0]
        r = jnp.where(l < nl - 1, jnp.maximum(r, 0.0), r)

        lanes = jax.lax.broadcasted_iota(jnp.int32, r.shape, 1)
        r_masked = jnp.where(lanes < n - d * bd, r, 0.0)

        @pl.when(l == 0)
        def _():
            xa_ref[:, pl.ds(d * bd, bd)] = r_masked

        @pl.when(l == 1)
        def _():
            xb_ref[:, pl.ds(d * bd, bd)] = r_masked

        @pl.when(l == nl - 1)
        def _():
            out_ref[...] = r.T


def _gcn_layers(wstack, bstack, nf, e_c, dinv_p, bs, bd):
    nl, h = wstack.shape[0], wstack.shape[1]
    n = e_c.shape[0]
    np_ = dinv_p.shape[1]
    nd, ns = np_ // bd, np_ // bs
    return pl.pallas_call(
        functools.partial(
            _gcn_body, n=n, nl=nl, nd=nd, ns=ns, bs=bs, bd=bd
        ),
        grid=(nl, nd, ns),
        in_specs=[
            pl.BlockSpec((1, h, h), lambda l, d, s: (l, 0, 0)),
            pl.BlockSpec(
                (bs, h), lambda l, d, s: (jnp.where(l == 0, s, 0), 0)
            ),
            pl.BlockSpec((bs, bd), lambda l, d, s: (s, d)),
            pl.BlockSpec((1, bs), lambda l, d, s: (0, s)),
            pl.BlockSpec((1, bd), lambda l, d, s: (0, d)),
            pl.BlockSpec((1, h, 1), lambda l, d, s: (l, 0, 0)),
        ],
        out_specs=pl.BlockSpec((bd, h), lambda l, d, s: (d, 0)),
        out_shape=jax.ShapeDtypeStruct((n, h), jnp.float32),
        scratch_shapes=[
            pltpu.VMEM((h, np_), jnp.float32),
            pltpu.VMEM((h, np_), jnp.float32),
            pltpu.VMEM((h, bd), jnp.float32),
            pltpu.VMEM((h, bd), _ETYPE),
        ],
    )(wstack, nf, e_c, dinv_p, dinv_p, bstack)


def kernel(node_features, edges, W1, b1, W2, b2, W3, b3):
    n, dim = node_features.shape
    bs = min(1280, -(-n // 128) * 128)
    np_ = -(-n // bs) * bs
    bd = min(4 * bs, np_)

    csum_bs = 1
    for c in range(16, min(n, 400) + 1, 16):
        if n % c == 0:
            csum_bs = c
    dinv, e_c = _colsum_dinv(edges, bs=csum_bs)

    dinv_p = jnp.pad(dinv, ((0, 0), (0, np_ - n)))
    wstack = jnp.stack([W1.T, W2.T, W3.T])
    bstack = jnp.stack([b1, b2, b3]).reshape(3, -1, 1)
    return _gcn_layers(wstack, bstack, node_features, e_c, dinv_p, bs, bd)
